# baseline (speedup 1.0000x reference)
import jax
import jax.numpy as jnp
from jax.experimental import pallas as pl

N = 100000
DIM = 128
BLK = 2000


def _body(m_ref, w_ref, x_ref, o_ref):
    x = x_ref[...]
    m = m_ref[...]
    w = w_ref[...]
    o_ref[...] = x * jnp.where(m > 0.0, w, 1.0)


def kernel(x, node_mask, deletion_weight):
    m = node_mask.astype(jnp.float32)[:, None]
    w = deletion_weight[None, :]
    return pl.pallas_call(
        _body,
        grid=(N // BLK,),
        in_specs=[
            pl.BlockSpec((BLK, 1), lambda i: (i, 0)),
            pl.BlockSpec((1, DIM), lambda i: (0, 0)),
            pl.BlockSpec((BLK, DIM), lambda i: (i, 0)),
        ],
        out_specs=pl.BlockSpec((BLK, DIM), lambda i: (i, 0)),
        out_shape=jax.ShapeDtypeStruct((N, DIM), jnp.float32),
    )(m, w, x)


# speedup vs baseline: 1.1424x; 1.1424x over previous
import jax
import jax.numpy as jnp
from jax.experimental import pallas as pl

N = 100000
DIM = 128
BLK = 10000


def _body(m_ref, w_ref, x_ref, o_ref):
    x = x_ref[...]
    m = m_ref[...]
    w = w_ref[...]
    o_ref[...] = x * jnp.where(m > 0.0, w, 1.0)


def kernel(x, node_mask, deletion_weight):
    m = node_mask.astype(jnp.float32)[:, None]
    w = deletion_weight[None, :]
    return pl.pallas_call(
        _body,
        grid=(N // BLK,),
        in_specs=[
            pl.BlockSpec((BLK, 1), lambda i: (i, 0)),
            pl.BlockSpec((1, DIM), lambda i: (0, 0)),
            pl.BlockSpec((BLK, DIM), lambda i: (i, 0)),
        ],
        out_specs=pl.BlockSpec((BLK, DIM), lambda i: (i, 0)),
        out_shape=jax.ShapeDtypeStruct((N, DIM), jnp.float32),
    )(m, w, x)
